# baseline (speedup 1.0000x reference)
import jax
import jax.numpy as jnp
from jax.experimental import pallas as pl
from jax.experimental.pallas import tpu as pltpu

_SPLIT = 104


def _embed_kernel(src_hbm, out_hbm, buf, sem_a, sem_b, sem_c, sem_d):
    n = src_hbm.shape[0]
    in0 = pltpu.make_async_copy(
        src_hbm.at[pl.ds(0, _SPLIT), :], buf.at[pl.ds(0, _SPLIT), :], sem_a
    )
    in1 = pltpu.make_async_copy(
        src_hbm.at[pl.ds(_SPLIT, n - _SPLIT), :],
        buf.at[pl.ds(_SPLIT, n - _SPLIT), :],
        sem_b,
    )
    out0 = pltpu.make_async_copy(
        buf.at[pl.ds(0, _SPLIT), :], out_hbm.at[0, pl.ds(0, _SPLIT), :], sem_c
    )
    out1 = pltpu.make_async_copy(
        buf.at[pl.ds(_SPLIT, n - _SPLIT), :],
        out_hbm.at[0, pl.ds(_SPLIT, n - _SPLIT), :],
        sem_d,
    )
    in0.start()
    in1.start()
    in0.wait()
    out0.start()
    in1.wait()
    out1.start()
    out0.wait()
    out1.wait()


def kernel(x, pos_table):
    maxlen = x.shape[-1]
    embed_dim = pos_table.shape[-1]
    return pl.pallas_call(
        _embed_kernel,
        in_specs=[pl.BlockSpec(memory_space=pl.ANY)],
        out_specs=pl.BlockSpec(memory_space=pl.ANY),
        out_shape=jax.ShapeDtypeStruct((1, maxlen, embed_dim), pos_table.dtype),
        scratch_shapes=[
            pltpu.VMEM((maxlen, embed_dim), pos_table.dtype),
            pltpu.SemaphoreType.DMA,
            pltpu.SemaphoreType.DMA,
            pltpu.SemaphoreType.DMA,
            pltpu.SemaphoreType.DMA,
        ],
    )(pos_table[:maxlen])


# speedup vs baseline: 1.0045x; 1.0045x over previous
import jax
import jax.numpy as jnp
from jax.experimental import pallas as pl
from jax.experimental.pallas import tpu as pltpu

def _embed_kernel(src_hbm, out_hbm, buf, sem):
    cin = pltpu.make_async_copy(src_hbm, buf, sem)
    cin.start()
    cin.wait()
    cout = pltpu.make_async_copy(buf, out_hbm.at[0], sem)
    cout.start()
    cout.wait()


def kernel(x, pos_table):
    maxlen = x.shape[-1]
    embed_dim = pos_table.shape[-1]
    return pl.pallas_call(
        _embed_kernel,
        in_specs=[pl.BlockSpec(memory_space=pl.ANY)],
        out_specs=pl.BlockSpec(memory_space=pl.ANY),
        out_shape=jax.ShapeDtypeStruct((1, maxlen, embed_dim), pos_table.dtype),
        scratch_shapes=[
            pltpu.VMEM((maxlen, embed_dim), pos_table.dtype),
            pltpu.SemaphoreType.DMA,
        ],
        compiler_params=pltpu.CompilerParams(
            skip_device_barrier=True,
            disable_bounds_checks=True,
            disable_semaphore_checks=True,
        ),
    )(pos_table[:maxlen])


# speedup vs baseline: 1.0105x; 1.0060x over previous
import jax
import jax.numpy as jnp
from jax.experimental import pallas as pl
from jax.experimental.pallas import tpu as pltpu


def _embed_kernel(src_hbm, out_hbm, buf, sem):
    cin = pltpu.make_async_copy(src_hbm, buf, sem)
    cin.start()
    cin.wait()
    cout = pltpu.make_async_copy(buf, out_hbm, sem)
    cout.start()
    cout.wait()


def kernel(x, pos_table):
    maxlen = x.shape[-1]
    embed_dim = pos_table.shape[-1]
    total = maxlen * embed_dim
    flat = pos_table[:maxlen].reshape(total)
    out = pl.pallas_call(
        _embed_kernel,
        in_specs=[pl.BlockSpec(memory_space=pl.ANY)],
        out_specs=pl.BlockSpec(memory_space=pl.ANY),
        out_shape=jax.ShapeDtypeStruct((total,), pos_table.dtype),
        scratch_shapes=[
            pltpu.VMEM((total,), pos_table.dtype),
            pltpu.SemaphoreType.DMA,
        ],
    )(flat)
    return out.reshape(1, maxlen, embed_dim)
